# baseline (speedup 1.0000x reference)
import functools

import jax
import jax.numpy as jnp
from jax import lax
from jax.experimental import pallas as pl
from jax.experimental.pallas import tpu as pltpu
from jax.experimental.pallas import tpu_sc as plsc

N_EXP = 8
TOPK = 2
H = 2048
I = 768
T = 8192
TM = 256
G = T * TOPK // TM + N_EXP
P = G * TM
TT = 256


def _routing(router_logits):
    m1 = jnp.max(router_logits, axis=-1)
    a1 = jnp.argmax(router_logits, axis=-1).astype(jnp.int32)
    oh1 = jax.nn.one_hot(a1, N_EXP, dtype=jnp.bool_)
    masked = jnp.where(oh1, -jnp.inf, router_logits)
    m2 = jnp.max(masked, axis=-1)
    a2 = jnp.argmax(masked, axis=-1).astype(jnp.int32)
    w0 = jax.lax.logistic(m1 - m2)
    w1 = 1.0 - w0
    e_flat = jnp.stack([a1, a2], axis=1).reshape(-1)
    oh = jax.nn.one_hot(e_flat, N_EXP, dtype=jnp.int32)
    csum = jnp.cumsum(oh, axis=0)
    rank = jnp.sum(oh * csum, axis=1) - 1
    cnt = csum[-1]
    pg = ((cnt + TM - 1) // TM) * TM
    starts = jnp.concatenate(
        [jnp.zeros((1,), jnp.int32), jnp.cumsum(pg)[:-1].astype(jnp.int32)])
    p = jnp.sum(oh * starts[None, :], axis=1) + rank
    tile_start = jnp.arange(G, dtype=jnp.int32) * TM
    eid = jnp.clip(
        jnp.searchsorted(starts, tile_start, side="right") - 1,
        0, N_EXP - 1).astype(jnp.int32)
    active = (tile_start < (starts[eid] + cnt[eid])).astype(jnp.int32)
    return w0, w1, p.astype(jnp.int32), eid, active


def _ffn_body(eid_ref, act_ref, x_ref, wg_ref, wu_ref, wd_ref, o_ref):
    i = pl.program_id(0)

    @pl.when(act_ref[i] == 1)
    def _():
        x = x_ref[...]
        g = jnp.dot(x, wg_ref[0], preferred_element_type=jnp.float32)
        u = jnp.dot(x, wu_ref[0], preferred_element_type=jnp.float32)
        a = (g * jax.lax.logistic(g) * u).astype(jnp.bfloat16)
        o_ref[...] = jnp.dot(
            a, wd_ref[0], preferred_element_type=jnp.float32
        ).astype(jnp.bfloat16)


def _ffn(eid, active, xs, wg, wu, wd):
    grid_spec = pltpu.PrefetchScalarGridSpec(
        num_scalar_prefetch=2,
        grid=(G,),
        in_specs=[
            pl.BlockSpec((TM, H), lambda i, eid, act: (i, 0)),
            pl.BlockSpec((1, H, I), lambda i, eid, act: (eid[i], 0, 0)),
            pl.BlockSpec((1, H, I), lambda i, eid, act: (eid[i], 0, 0)),
            pl.BlockSpec((1, I, H), lambda i, eid, act: (eid[i], 0, 0)),
        ],
        out_specs=pl.BlockSpec((TM, H), lambda i, eid, act: (i, 0)),
    )
    return pl.pallas_call(
        _ffn_body,
        grid_spec=grid_spec,
        out_shape=jax.ShapeDtypeStruct((P, H), jnp.bfloat16),
    )(eid, active, xs, wg, wu, wd)


def _combine_body(c_ref, w_ref, o_ref):
    c = c_ref[...].astype(jnp.float32) * w_ref[:, 0:1]
    cr = c.reshape(TT, TOPK, H)
    o_ref[...] = cr[:, 0, :] + cr[:, 1, :]


def _combine(c, w8):
    return pl.pallas_call(
        _combine_body,
        grid=(T // TT,),
        in_specs=[
            pl.BlockSpec((TOPK * TT, H), lambda i: (i, 0)),
            pl.BlockSpec((TOPK * TT, N_EXP), lambda i: (i, 0)),
        ],
        out_specs=pl.BlockSpec((TT, H), lambda i: (i, 0)),
        out_shape=jax.ShapeDtypeStruct((T, H), jnp.float32),
    )(c, w8)


NC = 2
NS = 16
NW = NC * NS
SL = H // 256

_SC_MESH = plsc.VectorSubcoreMesh(core_axis_name="c", subcore_axis_name="s")


def _to_i32_rows(x):
    r = x.shape[0]
    return jax.lax.bitcast_convert_type(
        x.reshape(r, SL * 128, 2), jnp.int32).reshape(r, SL, 128)


def _from_i32_rows(x3):
    r = x3.shape[0]
    return jax.lax.bitcast_convert_type(
        x3.reshape(r, SL * 128), jnp.bfloat16).reshape(r, H)

DISP_RB = 64
COMB_RB = 64


def _sc_dispatch(hid3, p0, p1):
    @functools.partial(
        pl.kernel,
        out_type=jax.ShapeDtypeStruct((P, SL, 128), jnp.int32),
        mesh=_SC_MESH,
        scratch_types=[
            pltpu.VMEM((DISP_RB, SL, 128), jnp.int32),
            pltpu.VMEM((DISP_RB,), jnp.int32),
            pltpu.VMEM((DISP_RB,), jnp.int32),
            pltpu.SemaphoreType.DMA,
        ],
    )
    def k(hid_hbm, p0_hbm, p1_hbm, xs_hbm, rows_v, i0_v, i1_v, sem):
        wid = lax.axis_index("s") * NC + lax.axis_index("c")
        base = wid * (T // NW)

        @pl.loop(0, T // NW, step=DISP_RB)
        def _(j):
            tok = base + j
            pltpu.sync_copy(hid_hbm.at[pl.ds(tok, DISP_RB)], rows_v)
            pltpu.sync_copy(p0_hbm.at[pl.ds(tok, DISP_RB)], i0_v)
            pltpu.sync_copy(p1_hbm.at[pl.ds(tok, DISP_RB)], i1_v)
            a = pltpu.async_copy(rows_v, xs_hbm.at[i0_v], sem)
            b = pltpu.async_copy(rows_v, xs_hbm.at[i1_v], sem)
            a.wait()
            b.wait()

    return k(hid3, p0, p1)


def _sc_combine_gather(d3, p_all):
    @functools.partial(
        pl.kernel,
        out_type=jax.ShapeDtypeStruct((TOPK * T, SL, 128), jnp.int32),
        mesh=_SC_MESH,
        scratch_types=[
            pltpu.VMEM((COMB_RB, SL, 128), jnp.int32),
            pltpu.VMEM((COMB_RB,), jnp.int32),
            pltpu.SemaphoreType.DMA,
        ],
    )
    def k(d_hbm, p_hbm, c_hbm, rows_v, idx_v, sem):
        wid = lax.axis_index("s") * NC + lax.axis_index("c")
        base = wid * (TOPK * T // NW)

        @pl.loop(0, TOPK * T // NW, step=COMB_RB)
        def _(j):
            row = base + j
            pltpu.sync_copy(p_hbm.at[pl.ds(row, COMB_RB)], idx_v)
            pltpu.async_copy(d_hbm.at[idx_v], rows_v, sem).wait()
            pltpu.sync_copy(rows_v, c_hbm.at[pl.ds(row, COMB_RB)])

    return k(d3, p_all)


def kernel(hidden_states, router_logits, gate_proj, up_proj, down_proj):
    w0, w1, p, eid, active = _routing(router_logits)
    hid_bf = hidden_states.astype(jnp.bfloat16)
    wg = gate_proj.astype(jnp.bfloat16)
    wu = up_proj.astype(jnp.bfloat16)
    wd = down_proj.astype(jnp.bfloat16)
    hid3 = _to_i32_rows(hid_bf)
    xs = _from_i32_rows(_sc_dispatch(hid3, p[0::2], p[1::2]))
    d = _ffn(eid, active, xs, wg, wu, wd)
    c = _from_i32_rows(_sc_combine_gather(_to_i32_rows(d), p))
    w8 = jnp.broadcast_to(
        jnp.stack([w0, w1], axis=1).reshape(-1, 1), (TOPK * T, N_EXP))
    return _combine(c, w8)


# speedup vs baseline: 2.9834x; 2.9834x over previous
import functools

import jax
import jax.numpy as jnp
from jax import lax
from jax.experimental import pallas as pl
from jax.experimental.pallas import tpu as pltpu
from jax.experimental.pallas import tpu_sc as plsc

N_EXP = 8
TOPK = 2
H = 2048
I = 768
T = 8192
TM = 256
G = T * TOPK // TM + N_EXP
P = G * TM
TT = 256


def _routing(router_logits):
    m1 = jnp.max(router_logits, axis=-1)
    a1 = jnp.argmax(router_logits, axis=-1).astype(jnp.int32)
    oh1 = jax.nn.one_hot(a1, N_EXP, dtype=jnp.bool_)
    masked = jnp.where(oh1, -jnp.inf, router_logits)
    m2 = jnp.max(masked, axis=-1)
    a2 = jnp.argmax(masked, axis=-1).astype(jnp.int32)
    w0 = jax.lax.logistic(m1 - m2)
    w1 = 1.0 - w0
    e_flat = jnp.stack([a1, a2], axis=1).reshape(-1)
    oh = jax.nn.one_hot(e_flat, N_EXP, dtype=jnp.int32)
    csum = jnp.cumsum(oh, axis=0)
    rank = jnp.sum(oh * csum, axis=1) - 1
    cnt = csum[-1]
    pg = ((cnt + TM - 1) // TM) * TM
    starts = jnp.concatenate(
        [jnp.zeros((1,), jnp.int32), jnp.cumsum(pg)[:-1].astype(jnp.int32)])
    p = jnp.sum(oh * starts[None, :], axis=1) + rank
    tile_start = jnp.arange(G, dtype=jnp.int32) * TM
    eid = jnp.clip(
        jnp.searchsorted(starts, tile_start, side="right") - 1,
        0, N_EXP - 1).astype(jnp.int32)
    active = (tile_start < (starts[eid] + cnt[eid])).astype(jnp.int32)
    return w0, w1, p.astype(jnp.int32), eid, active


def _ffn_body(eid_ref, act_ref, x_ref, wg_ref, wu_ref, wd_ref, o_ref):
    i = pl.program_id(0)

    @pl.when(act_ref[i] == 1)
    def _():
        x = x_ref[...].astype(jnp.bfloat16)
        g = jnp.dot(x, wg_ref[0], preferred_element_type=jnp.float32)
        u = jnp.dot(x, wu_ref[0], preferred_element_type=jnp.float32)
        a = (g * jax.lax.logistic(g) * u).astype(jnp.bfloat16)
        o_ref[...] = jnp.dot(a, wd_ref[0], preferred_element_type=jnp.float32)


def _ffn(eid, active, xs, wg, wu, wd):
    grid_spec = pltpu.PrefetchScalarGridSpec(
        num_scalar_prefetch=2,
        grid=(G,),
        in_specs=[
            pl.BlockSpec((TM, H), lambda i, eid, act: (i, 0)),
            pl.BlockSpec((1, H, I), lambda i, eid, act: (eid[i], 0, 0)),
            pl.BlockSpec((1, H, I), lambda i, eid, act: (eid[i], 0, 0)),
            pl.BlockSpec((1, I, H), lambda i, eid, act: (eid[i], 0, 0)),
        ],
        out_specs=pl.BlockSpec((TM, H), lambda i, eid, act: (i, 0)),
    )
    return pl.pallas_call(
        _ffn_body,
        grid_spec=grid_spec,
        out_shape=jax.ShapeDtypeStruct((P, H), jnp.float32),
    )(eid, active, xs, wg, wu, wd)


def _combine_body(c_ref, w_ref, o_ref):
    c = c_ref[...].astype(jnp.float32) * w_ref[:, 0:1]
    cr = c.reshape(TT, TOPK, H)
    o_ref[...] = cr[:, 0, :] + cr[:, 1, :]


def _combine(c, w8):
    return pl.pallas_call(
        _combine_body,
        grid=(T // TT,),
        in_specs=[
            pl.BlockSpec((TOPK * TT, H), lambda i: (i, 0)),
            pl.BlockSpec((TOPK * TT, N_EXP), lambda i: (i, 0)),
        ],
        out_specs=pl.BlockSpec((TT, H), lambda i: (i, 0)),
        out_shape=jax.ShapeDtypeStruct((T, H), jnp.float32),
    )(c, w8)


NC = 2
NS = 16
NW = NC * NS
SL = H // 128

_SC_MESH = plsc.VectorSubcoreMesh(core_axis_name="c", subcore_axis_name="s")

DISP_RB = 32
COMB_RB = 32


def _sc_dispatch(hid3, p0, p1):
    @functools.partial(
        pl.kernel,
        out_type=jax.ShapeDtypeStruct((P, SL, 128), jnp.float32),
        mesh=_SC_MESH,
        scratch_types=[
            pltpu.VMEM((DISP_RB, SL, 128), jnp.float32),
            pltpu.VMEM((DISP_RB,), jnp.int32),
            pltpu.VMEM((DISP_RB,), jnp.int32),
            pltpu.SemaphoreType.DMA,
        ],
    )
    def k(hid_hbm, p0_hbm, p1_hbm, xs_hbm, rows_v, i0_v, i1_v, sem):
        wid = lax.axis_index("s") * NC + lax.axis_index("c")
        base = wid * (T // NW)

        @pl.loop(0, T // NW, step=DISP_RB)
        def _(j):
            tok = base + j
            pltpu.sync_copy(hid_hbm.at[pl.ds(tok, DISP_RB)], rows_v)
            pltpu.sync_copy(p0_hbm.at[pl.ds(tok, DISP_RB)], i0_v)
            pltpu.sync_copy(p1_hbm.at[pl.ds(tok, DISP_RB)], i1_v)
            a = pltpu.async_copy(rows_v, xs_hbm.at[i0_v], sem)
            b = pltpu.async_copy(rows_v, xs_hbm.at[i1_v], sem)
            a.wait()
            b.wait()

    return k(hid3, p0, p1)


def _sc_combine_gather(d3, p_all):
    @functools.partial(
        pl.kernel,
        out_type=jax.ShapeDtypeStruct((TOPK * T, SL, 128), jnp.float32),
        mesh=_SC_MESH,
        scratch_types=[
            pltpu.VMEM((COMB_RB, SL, 128), jnp.float32),
            pltpu.VMEM((COMB_RB,), jnp.int32),
            pltpu.SemaphoreType.DMA,
        ],
    )
    def k(d_hbm, p_hbm, c_hbm, rows_v, idx_v, sem):
        wid = lax.axis_index("s") * NC + lax.axis_index("c")
        base = wid * (TOPK * T // NW)

        @pl.loop(0, TOPK * T // NW, step=COMB_RB)
        def _(j):
            row = base + j
            pltpu.sync_copy(p_hbm.at[pl.ds(row, COMB_RB)], idx_v)
            pltpu.async_copy(d_hbm.at[idx_v], rows_v, sem).wait()
            pltpu.sync_copy(rows_v, c_hbm.at[pl.ds(row, COMB_RB)])

    return k(d3, p_all)


def kernel(hidden_states, router_logits, gate_proj, up_proj, down_proj):
    w0, w1, p, eid, active = _routing(router_logits)
    wg = gate_proj.astype(jnp.bfloat16)
    wu = up_proj.astype(jnp.bfloat16)
    wd = down_proj.astype(jnp.bfloat16)
    hid3 = hidden_states.reshape(T, SL, 128)
    xs = _sc_dispatch(hid3, p[0::2], p[1::2]).reshape(P, H)
    d = _ffn(eid, active, xs, wg, wu, wd)
    c = _sc_combine_gather(d.reshape(P, SL, 128), p).reshape(TOPK * T, H)
    w8 = jnp.broadcast_to(
        jnp.stack([w0, w1], axis=1).reshape(-1, 1), (TOPK * T, N_EXP))
    return _combine(c, w8)


# speedup vs baseline: 4.7066x; 1.5776x over previous
import functools

import jax
import jax.numpy as jnp
from jax import lax
from jax.experimental import pallas as pl
from jax.experimental.pallas import tpu as pltpu
from jax.experimental.pallas import tpu_sc as plsc

N_EXP = 8
TOPK = 2
H = 2048
I = 768
T = 8192
TM = 256
G = T * TOPK // TM + N_EXP
P = G * TM
TT = 256


def _routing(router_logits):
    m1 = jnp.max(router_logits, axis=-1)
    a1 = jnp.argmax(router_logits, axis=-1).astype(jnp.int32)
    oh1 = jax.nn.one_hot(a1, N_EXP, dtype=jnp.bool_)
    masked = jnp.where(oh1, -jnp.inf, router_logits)
    m2 = jnp.max(masked, axis=-1)
    a2 = jnp.argmax(masked, axis=-1).astype(jnp.int32)
    w0 = jax.lax.logistic(m1 - m2)
    w1 = 1.0 - w0
    e_flat = jnp.stack([a1, a2], axis=1).reshape(-1)
    oh = jax.nn.one_hot(e_flat, N_EXP, dtype=jnp.int32)
    csum = jnp.cumsum(oh, axis=0)
    rank = jnp.sum(oh * csum, axis=1) - 1
    cnt = csum[-1]
    pg = ((cnt + TM - 1) // TM) * TM
    starts = jnp.concatenate(
        [jnp.zeros((1,), jnp.int32), jnp.cumsum(pg)[:-1].astype(jnp.int32)])
    p = jnp.sum(oh * starts[None, :], axis=1) + rank
    tile_start = jnp.arange(G, dtype=jnp.int32) * TM
    eid = jnp.clip(
        jnp.searchsorted(starts, tile_start, side="right") - 1,
        0, N_EXP - 1).astype(jnp.int32)
    active = (tile_start < (starts[eid] + cnt[eid])).astype(jnp.int32)
    return w0, w1, p.astype(jnp.int32), eid, active


def _ffn_body(eid_ref, act_ref, x_ref, wg_ref, wu_ref, wd_ref, o_ref):
    i = pl.program_id(0)

    @pl.when(act_ref[i] == 1)
    def _():
        x = x_ref[...].astype(jnp.bfloat16)
        g = jnp.dot(x, wg_ref[0], preferred_element_type=jnp.float32)
        u = jnp.dot(x, wu_ref[0], preferred_element_type=jnp.float32)
        a = (g * jax.lax.logistic(g) * u).astype(jnp.bfloat16)
        o_ref[...] = jnp.dot(a, wd_ref[0], preferred_element_type=jnp.float32)


def _ffn(eid, active, xs, wg, wu, wd):
    grid_spec = pltpu.PrefetchScalarGridSpec(
        num_scalar_prefetch=2,
        grid=(G,),
        in_specs=[
            pl.BlockSpec((TM, H), lambda i, eid, act: (i, 0)),
            pl.BlockSpec((1, H, I), lambda i, eid, act: (eid[i], 0, 0)),
            pl.BlockSpec((1, H, I), lambda i, eid, act: (eid[i], 0, 0)),
            pl.BlockSpec((1, I, H), lambda i, eid, act: (eid[i], 0, 0)),
        ],
        out_specs=pl.BlockSpec((TM, H), lambda i, eid, act: (i, 0)),
    )
    return pl.pallas_call(
        _ffn_body,
        grid_spec=grid_spec,
        out_shape=jax.ShapeDtypeStruct((P, H), jnp.float32),
    )(eid, active, xs, wg, wu, wd)


def _combine_body(c_ref, w_ref, o_ref):
    c = c_ref[...].astype(jnp.float32) * w_ref[:, 0:1]
    cr = c.reshape(TT, TOPK, H)
    o_ref[...] = cr[:, 0, :] + cr[:, 1, :]


def _combine(c, w8):
    return pl.pallas_call(
        _combine_body,
        grid=(T // TT,),
        in_specs=[
            pl.BlockSpec((TOPK * TT, H), lambda i: (i, 0)),
            pl.BlockSpec((TOPK * TT, N_EXP), lambda i: (i, 0)),
        ],
        out_specs=pl.BlockSpec((TT, H), lambda i: (i, 0)),
        out_shape=jax.ShapeDtypeStruct((T, H), jnp.float32),
    )(c, w8)


NC = 2
NS = 16
NW = NC * NS
SL = H // 128

_SC_MESH = plsc.VectorSubcoreMesh(core_axis_name="c", subcore_axis_name="s")

DISP_RB = 32
COMB_RB = 32


def _sc_dispatch(hid3, p0, p1):
    @functools.partial(
        pl.kernel,
        out_type=jax.ShapeDtypeStruct((P, H), jnp.float32),
        mesh=_SC_MESH,
        scratch_types=[
            pltpu.VMEM((DISP_RB, H), jnp.float32),
            pltpu.VMEM((DISP_RB,), jnp.int32),
            pltpu.VMEM((DISP_RB,), jnp.int32),
            pltpu.SemaphoreType.DMA,
        ],
    )
    def k(hid_hbm, p0_hbm, p1_hbm, xs_hbm, rows_v, i0_v, i1_v, sem):
        wid = lax.axis_index("s") * NC + lax.axis_index("c")
        base = wid * (T // NW)

        @pl.loop(0, T // NW, step=DISP_RB)
        def _(j):
            tok = base + j
            pltpu.sync_copy(hid_hbm.at[pl.ds(tok, DISP_RB)], rows_v)
            pltpu.sync_copy(p0_hbm.at[pl.ds(tok, DISP_RB)], i0_v)
            pltpu.sync_copy(p1_hbm.at[pl.ds(tok, DISP_RB)], i1_v)
            a = pltpu.async_copy(rows_v, xs_hbm.at[i0_v], sem)
            b = pltpu.async_copy(rows_v, xs_hbm.at[i1_v], sem)
            a.wait()
            b.wait()

    return k(hid3, p0, p1)


def _sc_combine_gather(d3, p_all):
    @functools.partial(
        pl.kernel,
        out_type=jax.ShapeDtypeStruct((TOPK * T, H), jnp.float32),
        mesh=_SC_MESH,
        scratch_types=[
            pltpu.VMEM((COMB_RB, H), jnp.float32),
            pltpu.VMEM((COMB_RB,), jnp.int32),
            pltpu.SemaphoreType.DMA,
        ],
    )
    def k(d_hbm, p_hbm, c_hbm, rows_v, idx_v, sem):
        wid = lax.axis_index("s") * NC + lax.axis_index("c")
        base = wid * (TOPK * T // NW)

        @pl.loop(0, TOPK * T // NW, step=COMB_RB)
        def _(j):
            row = base + j
            pltpu.sync_copy(p_hbm.at[pl.ds(row, COMB_RB)], idx_v)
            pltpu.async_copy(d_hbm.at[idx_v], rows_v, sem).wait()
            pltpu.sync_copy(rows_v, c_hbm.at[pl.ds(row, COMB_RB)])

    return k(d3, p_all)


def kernel(hidden_states, router_logits, gate_proj, up_proj, down_proj):
    w0, w1, p, eid, active = _routing(router_logits)
    wg = gate_proj.astype(jnp.bfloat16)
    wu = up_proj.astype(jnp.bfloat16)
    wd = down_proj.astype(jnp.bfloat16)
    xs = _sc_dispatch(hidden_states, p[0::2], p[1::2])
    d = _ffn(eid, active, xs, wg, wu, wd)
    c = _sc_combine_gather(d, p)
    w8 = jnp.broadcast_to(
        jnp.stack([w0, w1], axis=1).reshape(-1, 1), (TOPK * T, N_EXP))
    return _combine(c, w8)
